# baseline (speedup 1.0000x reference)
import jax
import jax.numpy as jnp
import numpy as np
from math import log
from jax.experimental import pallas as pl
from jax.experimental.pallas import tpu as pltpu

_INPUT_DIM = 128
_RULE_NUM = 64
_TAU_0 = 0.75
_EPS = float(np.finfo(np.float32).tiny)
_TAU = _TAU_0 * (1.0 / (_RULE_NUM + _EPS)) ** (1.0 / _INPUT_DIM)

_TB = 128


def _fuzzy_body(q_ref, kT_ref, sT_ref, v_ref, out_ref, loss_ref, imp_ref):
    b = pl.program_id(0)
    t = pl.program_id(1)
    step = b * pl.num_programs(1) + t
    nsteps = pl.num_programs(0) * pl.num_programs(1)

    q = q_ref[0, :, 0, :]
    inv = 1.0 / (sT_ref[...] + _EPS)
    k = kT_ref[...]
    cols = []
    for r in range(_RULE_NUM):
        dr = (q - k[r:r + 1, :]) * inv[r:r + 1, :]
        cols.append(jnp.sum(dr * dr, axis=1, keepdims=True))
    d2 = jnp.concatenate(cols, axis=1)
    fire = jnp.exp(d2 * (-0.5 / _INPUT_DIM)) + _EPS
    nfs = fire / (jnp.sum(fire, axis=1, keepdims=True) + _EPS)

    excess = jnp.zeros_like(nfs)
    for j in range(_RULE_NUM):
        cj = nfs[:, j:j + 1]
        excess = excess + jnp.where(cj > nfs, cj, 0.0)
    keep = (excess < _TAU).astype(jnp.float32)
    kept = nfs * keep
    nfs2 = kept / (jnp.sum(kept, axis=1, keepdims=True) + _EPS)

    pred = jnp.zeros_like(q)
    for r in range(_RULE_NUM):
        pred = pred + v_ref[0, :, r, :] * nfs2[:, r:r + 1]
    out_ref[0] = pred.T

    blk_imp = jnp.sum(nfs2, axis=0, keepdims=True)

    @pl.when(step == 0)
    def _():
        imp_ref[...] = blk_imp

    @pl.when(step != 0)
    def _():
        imp_ref[...] = imp_ref[...] + blk_imp

    @pl.when(step == nsteps - 1)
    def _():
        imp = imp_ref[...]
        mean = jnp.sum(imp, axis=1, keepdims=True) / _RULE_NUM
        var = jnp.sum((imp - mean) ** 2, axis=1, keepdims=True) / (_RULE_NUM - 1)
        loss_ref[...] = var / (mean * mean + 1e-10)


def kernel(queries, keys, values, sigma):
    B, T = queries.shape[0], queries.shape[1]
    D, R = _INPUT_DIM, _RULE_NUM
    grid = (B, T // _TB)
    pred, loss = pl.pallas_call(
        _fuzzy_body,
        grid=grid,
        in_specs=[
            pl.BlockSpec((1, _TB, 1, D), lambda b, t: (b, t, 0, 0)),
            pl.BlockSpec((R, D), lambda b, t: (0, 0)),
            pl.BlockSpec((R, D), lambda b, t: (0, 0)),
            pl.BlockSpec((1, _TB, R, D), lambda b, t: (b, t, 0, 0)),
        ],
        out_specs=[
            pl.BlockSpec((1, D, _TB), lambda b, t: (b, 0, t)),
            pl.BlockSpec((1, 1), lambda b, t: (0, 0)),
        ],
        out_shape=[
            jax.ShapeDtypeStruct((B, D, T), jnp.float32),
            jax.ShapeDtypeStruct((1, 1), jnp.float32),
        ],
        scratch_shapes=[pltpu.VMEM((1, R), jnp.float32)],
    )(queries, keys, sigma, values)
    return pred, loss.reshape(())


# speedup vs baseline: 1.0148x; 1.0148x over previous
import jax
import jax.numpy as jnp
import numpy as np
from jax.experimental import pallas as pl
from jax.experimental.pallas import tpu as pltpu

_D = 128
_R = 64
_TAU_0 = 0.75
_EPS = float(np.finfo(np.float32).tiny)
_TAU = _TAU_0 * (1.0 / (_R + _EPS)) ** (1.0 / _D)

_TB = 128
_RD = _R * _D


def _tile_mat():
    m = np.zeros((_D, _RD), np.float32)
    for r in range(_R):
        m[np.arange(_D), r * _D + np.arange(_D)] = 1.0
    return m


def _bcast_mat():
    m = np.zeros((_R, _RD), np.float32)
    for r in range(_R):
        m[r, r * _D:(r + 1) * _D] = 1.0
    return m


def _bcast64_mat():
    m = np.zeros((_R, _R * _R), np.float32)
    for j in range(_R):
        m[j, j * _R:(j + 1) * _R] = 1.0
    return m


def _sum_mat():
    m = np.zeros((_RD, _D), np.float32)
    for r in range(_R):
        m[r * _D + np.arange(_D), np.arange(_D)] = 1.0
    return m


_TILE = _tile_mat()
_BCAST = _bcast_mat()
_BCAST64 = _bcast64_mat()
_SUM = _sum_mat()


def _fuzzy_body(q_ref, kw_ref, iw_ref, v_ref, tile_ref, bc_ref, bc64_ref,
                sum_ref, out_ref, loss_ref, imp_ref):
    step = pl.program_id(0) * pl.num_programs(1) + pl.program_id(1)
    nsteps = pl.num_programs(0) * pl.num_programs(1)

    q = q_ref[0, :, 0, :]
    qw = jnp.dot(q, tile_ref[...], preferred_element_type=jnp.float32,
                 precision=jax.lax.Precision.HIGHEST)
    dist = (qw - kw_ref[...]) * iw_ref[...]
    dsq = dist * dist
    cols = [jnp.sum(dsq[:, r * _D:(r + 1) * _D], axis=1, keepdims=True)
            for r in range(_R)]
    d2 = jnp.concatenate(cols, axis=1)
    fire = jnp.exp(d2 * (-0.5 / _D)) + _EPS
    nfs = fire / (jnp.sum(fire, axis=1, keepdims=True) + _EPS)

    nall = jnp.dot(nfs, bc64_ref[...], preferred_element_type=jnp.float32,
                   precision=jax.lax.Precision.HIGHEST)
    excess = jnp.zeros_like(nfs)
    for j in range(_R):
        cj = nall[:, j * _R:(j + 1) * _R]
        excess = excess + jnp.where(cj > nfs, cj, 0.0)
    keep = (excess < _TAU).astype(jnp.float32)
    kept = nfs * keep
    nfs2 = kept / (jnp.sum(kept, axis=1, keepdims=True) + _EPS)

    wall = jnp.dot(nfs2, bc_ref[...], preferred_element_type=jnp.float32)
    prod = v_ref[0] * wall
    pred = jnp.dot(prod, sum_ref[...], preferred_element_type=jnp.float32)
    out_ref[0] = pred.T

    blk_imp = jnp.sum(nfs2, axis=0, keepdims=True)

    @pl.when(step == 0)
    def _():
        imp_ref[...] = blk_imp

    @pl.when(step != 0)
    def _():
        imp_ref[...] = imp_ref[...] + blk_imp

    @pl.when(step == nsteps - 1)
    def _():
        imp = imp_ref[...]
        mean = jnp.sum(imp, axis=1, keepdims=True) / _R
        var = jnp.sum((imp - mean) ** 2, axis=1, keepdims=True) / (_R - 1)
        loss_ref[...] = var / (mean * mean + 1e-10)


def kernel(queries, keys, values, sigma):
    B, T = queries.shape[0], queries.shape[1]
    v2 = values.reshape(B, T, _RD)
    kw = keys.reshape(1, _RD)
    iw = (1.0 / (sigma + _EPS)).reshape(1, _RD)
    grid = (B, T // _TB)
    const = lambda b, t: (0, 0)
    pred, loss = pl.pallas_call(
        _fuzzy_body,
        grid=grid,
        in_specs=[
            pl.BlockSpec((1, _TB, 1, _D), lambda b, t: (b, t, 0, 0)),
            pl.BlockSpec((1, _RD), const),
            pl.BlockSpec((1, _RD), const),
            pl.BlockSpec((1, _TB, _RD), lambda b, t: (b, t, 0)),
            pl.BlockSpec((_D, _RD), const),
            pl.BlockSpec((_R, _RD), const),
            pl.BlockSpec((_R, _R * _R), const),
            pl.BlockSpec((_RD, _D), const),
        ],
        out_specs=[
            pl.BlockSpec((1, _D, _TB), lambda b, t: (b, 0, t)),
            pl.BlockSpec((1, 1), const),
        ],
        out_shape=[
            jax.ShapeDtypeStruct((B, _D, T), jnp.float32),
            jax.ShapeDtypeStruct((1, 1), jnp.float32),
        ],
        scratch_shapes=[pltpu.VMEM((1, _R), jnp.float32)],
    )(queries, kw, iw, v2,
      jnp.asarray(_TILE), jnp.asarray(_BCAST), jnp.asarray(_BCAST64),
      jnp.asarray(_SUM))
    return pred, loss.reshape(())


# speedup vs baseline: 2.7751x; 2.7346x over previous
import functools

import jax
import jax.numpy as jnp
import numpy as np
from jax import lax
from jax.experimental import pallas as pl
from jax.experimental.pallas import tpu as pltpu
from jax.experimental.pallas import tpu_sc as plsc

_D = 128
_R = 64
_TAU_0 = 0.75
_EPS = float(np.finfo(np.float32).tiny)
_TAU = _TAU_0 * (1.0 / (_R + _EPS)) ** (1.0 / _D)

_TB = 128
_RD = _R * _D
_N = 4 * 2048
_NW = 32
_TPW = _N // _NW
_CAP = _TPW * _R + 16


def _weights_body(q_ref, kw_ref, iw_ref, out_ref, loss_ref, imp_ref):
    step = pl.program_id(0)
    nsteps = pl.num_programs(0)

    q = q_ref[:, 0, :]
    qw = jnp.concatenate([q] * _R, axis=1)
    dist = (qw - kw_ref[...]) * iw_ref[...]
    dsq = dist * dist
    cols = [jnp.sum(dsq[:, r * _D:(r + 1) * _D], axis=1, keepdims=True)
            for r in range(_R)]
    d2 = jnp.concatenate(cols, axis=1)
    fire = jnp.exp(d2 * (-0.5 / _D)) + _EPS
    nfs = fire / (jnp.sum(fire, axis=1, keepdims=True) + _EPS)

    excess = jnp.zeros_like(nfs)
    for j in range(_R):
        cj = nfs[:, j:j + 1]
        excess = excess + jnp.where(cj > nfs, cj, 0.0)
    keep = (excess < _TAU).astype(jnp.float32)
    kept = nfs * keep
    nfs2 = kept / (jnp.sum(kept, axis=1, keepdims=True) + _EPS)
    out_ref[...] = nfs2

    blk_imp = jnp.sum(nfs2, axis=0, keepdims=True)

    @pl.when(step == 0)
    def _():
        imp_ref[...] = blk_imp

    @pl.when(step != 0)
    def _():
        imp_ref[...] = imp_ref[...] + blk_imp

    @pl.when(step == nsteps - 1)
    def _():
        imp = imp_ref[...]
        mean = jnp.sum(imp, axis=1, keepdims=True) / _R
        var = jnp.sum((imp - mean) ** 2, axis=1, keepdims=True) / (_R - 1)
        loss_ref[...] = var / (mean * mean + 1e-10)


def _stage1(qflat, kw, iw):
    const = lambda t: (0, 0)
    return pl.pallas_call(
        _weights_body,
        grid=(_N // _TB,),
        in_specs=[
            pl.BlockSpec((_TB, 1, _D), lambda t: (t, 0, 0)),
            pl.BlockSpec((1, _RD), const),
            pl.BlockSpec((1, _RD), const),
        ],
        out_specs=[
            pl.BlockSpec((_TB, _R), lambda t: (t, 0)),
            pl.BlockSpec((1, 1), const),
        ],
        out_shape=[
            jax.ShapeDtypeStruct((_N, _R), jnp.float32),
            jax.ShapeDtypeStruct((1, 1), jnp.float32),
        ],
        scratch_shapes=[pltpu.VMEM((1, _R), jnp.float32)],
    )(qflat, kw, iw)


def _sc_body(w_hbm, v_hbm, out_hbm,
             wv, idxb, wkb, tidb, stage, zbuf, shared, sem):
    c = lax.axis_index("c")
    s = lax.axis_index("s")
    wid = c * 16 + s
    gbase = wid * _TPW
    lbase = s * _TPW

    pltpu.sync_copy(w_hbm.at[pl.ds(gbase * _R, _TPW * _R)], wv)

    for j in range(16):
        for k in range(8):
            zbuf[j, pl.ds(k * 16, 16)] = jnp.zeros((16,), jnp.float32)
    for blk in range(16):
        pltpu.sync_copy(zbuf, shared.at[pl.ds(lbase + blk * 16, 16)])

    lane = lax.iota(jnp.int32, 16)

    def _build(t, cnt):
        rowbase = (gbase + t) * _R
        for j in range(4):
            w = wv[pl.ds(t * _R + j * 16, 16)]
            m = w > 0.0
            idxv = rowbase + j * 16 + lane
            tidv = jnp.full((16,), lbase, jnp.int32) + t
            key = jnp.where(m, lane, lane + 16)
            idxb[pl.ds(cnt, 16)] = plsc.sort_key_val(key, idxv.astype(jnp.float32))[1]
            wkb[pl.ds(cnt, 16)] = plsc.sort_key_val(key, w)[1]
            tidb[pl.ds(cnt, 16)] = plsc.sort_key_val(key, tidv.astype(jnp.float32))[1]
            cnt = cnt + jnp.sum(m.astype(jnp.int32))
        return cnt
    cnt = lax.fori_loop(0, _TPW, _build, jnp.int32(0))

    idxb[pl.ds(cnt, 16)] = jnp.zeros((16,), jnp.float32)
    wkb[pl.ds(cnt, 16)] = jnp.zeros((16,), jnp.float32)
    tidb[pl.ds(cnt, 16)] = jnp.full((16,), lbase, jnp.int32).astype(jnp.float32)

    nchunk = (cnt + 15) // 16

    def _chunk(i, _):
        idx16 = idxb[pl.ds(i * 16, 16)].astype(jnp.int32)
        w16 = wkb[pl.ds(i * 16, 16)]
        tid16 = tidb[pl.ds(i * 16, 16)].astype(jnp.int32)
        pltpu.async_copy(v_hbm.at[idx16], stage, sem).wait()
        for j in range(16):
            wj = jnp.sum(jnp.where(lane == j, w16, 0.0))
            for k in range(8):
                stage[j, pl.ds(k * 16, 16)] = stage[j, pl.ds(k * 16, 16)] * wj
        pltpu.sync_copy(stage, shared.at[tid16], add=True)
        return 0
    lax.fori_loop(0, nchunk, _chunk, 0)

    plsc.subcore_barrier()
    pltpu.sync_copy(shared.at[pl.ds(lbase, _TPW)],
                    out_hbm.at[pl.ds(gbase, _TPW)])


def _stage2(nfs2, vrows):
    mesh = plsc.VectorSubcoreMesh(core_axis_name="c", subcore_axis_name="s")
    f = functools.partial(
        pl.kernel,
        mesh=mesh,
        compiler_params=pltpu.CompilerParams(needs_layout_passes=False),
        out_type=jax.ShapeDtypeStruct((_N, _D), jnp.float32),
        scratch_types=[
            pltpu.VMEM((_TPW * _R,), jnp.float32),
            pltpu.VMEM((_CAP,), jnp.float32),
            pltpu.VMEM((_CAP,), jnp.float32),
            pltpu.VMEM((_CAP,), jnp.float32),
            pltpu.VMEM((16, _D), jnp.float32),
            pltpu.VMEM((16, _D), jnp.float32),
            pltpu.VMEM_SHARED((_N // 2, _D), jnp.float32),
            pltpu.SemaphoreType.DMA,
        ],
    )
    return f(_sc_body)(nfs2.reshape(_N * _R), vrows)


def _transpose_body(p_ref, out_ref):
    out_ref[0] = p_ref[0].T


def _stage3(pred_rows, B, T):
    p3 = pred_rows.reshape(B, T, _D)
    return pl.pallas_call(
        _transpose_body,
        grid=(B, T // 512),
        in_specs=[pl.BlockSpec((1, 512, _D), lambda b, t: (b, t, 0))],
        out_specs=pl.BlockSpec((1, _D, 512), lambda b, t: (b, 0, t)),
        out_shape=jax.ShapeDtypeStruct((B, _D, T), jnp.float32),
    )(p3)


def kernel(queries, keys, values, sigma):
    B, T = queries.shape[0], queries.shape[1]
    qflat = queries.reshape(B * T, 1, _D)
    kw = keys.reshape(1, _RD)
    iw = (1.0 / (sigma + _EPS)).reshape(1, _RD)
    vrows = values.reshape(B * T * _R, _D)

    nfs2, loss = _stage1(qflat, kw, iw)
    pred_rows = _stage2(nfs2, vrows)
    pred = _stage3(pred_rows, B, T)
    return pred, loss.reshape(())


# speedup vs baseline: 2.9534x; 1.0642x over previous
import functools

import jax
import jax.numpy as jnp
import numpy as np
from jax import lax
from jax.experimental import pallas as pl
from jax.experimental.pallas import tpu as pltpu
from jax.experimental.pallas import tpu_sc as plsc

_D = 128
_R = 64
_TAU_0 = 0.75
_EPS = float(np.finfo(np.float32).tiny)
_TAU = _TAU_0 * (1.0 / (_R + _EPS)) ** (1.0 / _D)

_TB = 256
_RD = _R * _D
_N = 4 * 2048
_NW = 32
_TPW = _N // _NW
_CAP = _TPW * _R + 16


def _weights_body(q_ref, kw_ref, iw_ref, out_ref, loss_ref, imp_ref):
    step = pl.program_id(0)
    nsteps = pl.num_programs(0)

    q = q_ref[:, 0, :]
    qw = jnp.concatenate([q] * _R, axis=1)
    dist = (qw - kw_ref[...]) * iw_ref[...]
    dsq = dist * dist
    cols = [jnp.sum(dsq[:, r * _D:(r + 1) * _D], axis=1, keepdims=True)
            for r in range(_R)]
    d2 = jnp.concatenate(cols, axis=1)
    fire = jnp.exp(d2 * (-0.5 / _D)) + _EPS
    nfs = fire / (jnp.sum(fire, axis=1, keepdims=True) + _EPS)

    excess = jnp.zeros_like(nfs)
    for j in range(_R):
        cj = nfs[:, j:j + 1]
        excess = excess + jnp.where(cj > nfs, cj, 0.0)
    keep = (excess < _TAU).astype(jnp.float32)
    kept = nfs * keep
    nfs2 = kept / (jnp.sum(kept, axis=1, keepdims=True) + _EPS)
    out_ref[...] = nfs2

    blk_imp = jnp.sum(nfs2, axis=0, keepdims=True)

    @pl.when(step == 0)
    def _():
        imp_ref[...] = blk_imp

    @pl.when(step != 0)
    def _():
        imp_ref[...] = imp_ref[...] + blk_imp

    @pl.when(step == nsteps - 1)
    def _():
        imp = imp_ref[...]
        mean = jnp.sum(imp, axis=1, keepdims=True) / _R
        var = jnp.sum((imp - mean) ** 2, axis=1, keepdims=True) / (_R - 1)
        loss_ref[...] = var / (mean * mean + 1e-10)


def _stage1(qflat, kw, iw):
    const = lambda t: (0, 0)
    return pl.pallas_call(
        _weights_body,
        grid=(_N // _TB,),
        in_specs=[
            pl.BlockSpec((_TB, 1, _D), lambda t: (t, 0, 0)),
            pl.BlockSpec((1, _RD), const),
            pl.BlockSpec((1, _RD), const),
        ],
        out_specs=[
            pl.BlockSpec((_TB, _R), lambda t: (t, 0)),
            pl.BlockSpec((1, 1), const),
        ],
        out_shape=[
            jax.ShapeDtypeStruct((_N, _R), jnp.float32),
            jax.ShapeDtypeStruct((1, 1), jnp.float32),
        ],
        scratch_shapes=[pltpu.VMEM((1, _R), jnp.float32)],
    )(qflat, kw, iw)


def _sc_body(w_hbm, v_hbm, out_hbm,
             wv, idxb, wkb, tidb, stage, zbuf, shared, sem):
    c = lax.axis_index("c")
    s = lax.axis_index("s")
    wid = c * 16 + s
    gbase = wid * _TPW
    lbase = s * _TPW

    pltpu.sync_copy(w_hbm.at[pl.ds(gbase * _R, _TPW * _R)], wv)

    for j in range(16):
        for k in range(8):
            zbuf[j, pl.ds(k * 16, 16)] = jnp.zeros((16,), jnp.float32)
    for blk in range(16):
        pltpu.sync_copy(zbuf, shared.at[pl.ds(lbase + blk * 16, 16)])

    lane = lax.iota(jnp.int32, 16)

    def _build(t, cnt):
        rowbase = (gbase + t) * _R
        for j in range(4):
            w = wv[pl.ds(t * _R + j * 16, 16)]
            m = w > 0.0
            idxv = rowbase + j * 16 + lane
            tidv = jnp.full((16,), lbase, jnp.int32) + t
            key = jnp.where(m, lane, lane + 16)
            idxb[pl.ds(cnt, 16)] = plsc.sort_key_val(key, idxv.astype(jnp.float32))[1]
            wkb[pl.ds(cnt, 16)] = plsc.sort_key_val(key, w)[1]
            tidb[pl.ds(cnt, 16)] = plsc.sort_key_val(key, tidv.astype(jnp.float32))[1]
            cnt = cnt + jnp.sum(m.astype(jnp.int32))
        return cnt
    cnt = lax.fori_loop(0, _TPW, _build, jnp.int32(0))

    idxb[pl.ds(cnt, 16)] = jnp.zeros((16,), jnp.float32)
    wkb[pl.ds(cnt, 16)] = jnp.zeros((16,), jnp.float32)
    tidb[pl.ds(cnt, 16)] = jnp.full((16,), lbase, jnp.int32).astype(jnp.float32)

    nchunk = (cnt + 15) // 16

    def _chunk(i, _):
        idx16 = idxb[pl.ds(i * 16, 16)].astype(jnp.int32)
        w16 = wkb[pl.ds(i * 16, 16)]
        tid16 = tidb[pl.ds(i * 16, 16)].astype(jnp.int32)
        pltpu.async_copy(v_hbm.at[idx16], stage, sem).wait()
        for j in range(16):
            wj = jnp.sum(jnp.where(lane == j, w16, 0.0))
            for k in range(8):
                stage[j, pl.ds(k * 16, 16)] = stage[j, pl.ds(k * 16, 16)] * wj
        pltpu.sync_copy(stage, shared.at[tid16], add=True)
        return 0
    lax.fori_loop(0, nchunk, _chunk, 0)

    plsc.subcore_barrier()
    pltpu.sync_copy(shared.at[pl.ds(lbase, _TPW)],
                    out_hbm.at[pl.ds(gbase, _TPW)])


def _stage2(nfs2, vrows):
    mesh = plsc.VectorSubcoreMesh(core_axis_name="c", subcore_axis_name="s")
    f = functools.partial(
        pl.kernel,
        mesh=mesh,
        compiler_params=pltpu.CompilerParams(needs_layout_passes=False),
        out_type=jax.ShapeDtypeStruct((_N, _D), jnp.float32),
        scratch_types=[
            pltpu.VMEM((_TPW * _R,), jnp.float32),
            pltpu.VMEM((_CAP,), jnp.float32),
            pltpu.VMEM((_CAP,), jnp.float32),
            pltpu.VMEM((_CAP,), jnp.float32),
            pltpu.VMEM((16, _D), jnp.float32),
            pltpu.VMEM((16, _D), jnp.float32),
            pltpu.VMEM_SHARED((_N // 2, _D), jnp.float32),
            pltpu.SemaphoreType.DMA,
        ],
    )
    return f(_sc_body)(nfs2.reshape(_N * _R), vrows)


def _transpose_body(p_ref, out_ref):
    out_ref[0] = p_ref[0].T


def _stage3(pred_rows, B, T):
    p3 = pred_rows.reshape(B, T, _D)
    return pl.pallas_call(
        _transpose_body,
        grid=(B, T // 512),
        in_specs=[pl.BlockSpec((1, 512, _D), lambda b, t: (b, t, 0))],
        out_specs=pl.BlockSpec((1, _D, 512), lambda b, t: (b, 0, t)),
        out_shape=jax.ShapeDtypeStruct((B, _D, T), jnp.float32),
    )(p3)


def kernel(queries, keys, values, sigma):
    B, T = queries.shape[0], queries.shape[1]
    qflat = queries.reshape(B * T, 1, _D)
    kw = keys.reshape(1, _RD)
    iw = (1.0 / (sigma + _EPS)).reshape(1, _RD)
    vrows = values.reshape(B * T * _R, _D)

    nfs2, loss = _stage1(qflat, kw, iw)
    pred_rows = _stage2(nfs2, vrows)
    pred = _stage3(pred_rows, B, T)
    return pred, loss.reshape(())


# speedup vs baseline: 3.0112x; 1.0196x over previous
import functools

import jax
import jax.numpy as jnp
import numpy as np
from jax import lax
from jax.experimental import pallas as pl
from jax.experimental.pallas import tpu as pltpu
from jax.experimental.pallas import tpu_sc as plsc

_D = 128
_R = 64
_TAU_0 = 0.75
_EPS = float(np.finfo(np.float32).tiny)
_TAU = _TAU_0 * (1.0 / (_R + _EPS)) ** (1.0 / _D)

_TB = 512
_RD = _R * _D
_N = 4 * 2048
_NW = 32
_TPW = _N // _NW
_CAP = _TPW * _R + 16


def _weights_body(q_ref, kw_ref, iw_ref, out_ref, loss_ref, imp_ref):
    step = pl.program_id(0)
    nsteps = pl.num_programs(0)

    q = q_ref[:, 0, :]
    qw = jnp.concatenate([q] * _R, axis=1)
    dist = (qw - kw_ref[...]) * iw_ref[...]
    dsq = dist * dist
    cols = [jnp.sum(dsq[:, r * _D:(r + 1) * _D], axis=1, keepdims=True)
            for r in range(_R)]
    d2 = jnp.concatenate(cols, axis=1)
    fire = jnp.exp(d2 * (-0.5 / _D)) + _EPS
    nfs = fire / (jnp.sum(fire, axis=1, keepdims=True) + _EPS)

    excess = jnp.zeros_like(nfs)
    for j in range(_R):
        cj = nfs[:, j:j + 1]
        excess = excess + jnp.where(cj > nfs, cj, 0.0)
    keep = (excess < _TAU).astype(jnp.float32)
    kept = nfs * keep
    nfs2 = kept / (jnp.sum(kept, axis=1, keepdims=True) + _EPS)
    out_ref[...] = nfs2

    blk_imp = jnp.sum(nfs2, axis=0, keepdims=True)

    @pl.when(step == 0)
    def _():
        imp_ref[...] = blk_imp

    @pl.when(step != 0)
    def _():
        imp_ref[...] = imp_ref[...] + blk_imp

    @pl.when(step == nsteps - 1)
    def _():
        imp = imp_ref[...]
        mean = jnp.sum(imp, axis=1, keepdims=True) / _R
        var = jnp.sum((imp - mean) ** 2, axis=1, keepdims=True) / (_R - 1)
        loss_ref[...] = var / (mean * mean + 1e-10)


def _stage1(qflat, kw, iw):
    const = lambda t: (0, 0)
    return pl.pallas_call(
        _weights_body,
        grid=(_N // _TB,),
        in_specs=[
            pl.BlockSpec((_TB, 1, _D), lambda t: (t, 0, 0)),
            pl.BlockSpec((1, _RD), const),
            pl.BlockSpec((1, _RD), const),
        ],
        out_specs=[
            pl.BlockSpec((_TB, _R), lambda t: (t, 0)),
            pl.BlockSpec((1, 1), const),
        ],
        out_shape=[
            jax.ShapeDtypeStruct((_N, _R), jnp.float32),
            jax.ShapeDtypeStruct((1, 1), jnp.float32),
        ],
        scratch_shapes=[pltpu.VMEM((1, _R), jnp.float32)],
    )(qflat, kw, iw)


def _sc_body(w_hbm, v_hbm, out_hbm,
             wv, idxb, wkb, tidb, stage, zbuf, shared, sem):
    c = lax.axis_index("c")
    s = lax.axis_index("s")
    wid = c * 16 + s
    gbase = wid * _TPW
    lbase = s * _TPW

    pltpu.sync_copy(w_hbm.at[pl.ds(gbase * _R, _TPW * _R)], wv)

    for j in range(16):
        for k in range(8):
            zbuf[j, pl.ds(k * 16, 16)] = jnp.zeros((16,), jnp.float32)
    for blk in range(16):
        pltpu.sync_copy(zbuf, shared.at[pl.ds(lbase + blk * 16, 16)])

    lane = lax.iota(jnp.int32, 16)

    def _build(t, cnt):
        rowbase = (gbase + t) * _R
        for j in range(4):
            w = wv[pl.ds(t * _R + j * 16, 16)]
            m = w > 0.0
            idxv = rowbase + j * 16 + lane
            tidv = jnp.full((16,), lbase, jnp.int32) + t
            key = jnp.where(m, lane, lane + 16)
            idxb[pl.ds(cnt, 16)] = plsc.sort_key_val(key, idxv.astype(jnp.float32))[1]
            wkb[pl.ds(cnt, 16)] = plsc.sort_key_val(key, w)[1]
            tidb[pl.ds(cnt, 16)] = plsc.sort_key_val(key, tidv.astype(jnp.float32))[1]
            cnt = cnt + jnp.sum(m.astype(jnp.int32))
        return cnt
    cnt = lax.fori_loop(0, _TPW, _build, jnp.int32(0))

    idxb[pl.ds(cnt, 16)] = jnp.zeros((16,), jnp.float32)
    wkb[pl.ds(cnt, 16)] = jnp.zeros((16,), jnp.float32)
    tidb[pl.ds(cnt, 16)] = jnp.full((16,), lbase, jnp.int32).astype(jnp.float32)

    nchunk = (cnt + 15) // 16

    def _chunk(i, _):
        idx16 = idxb[pl.ds(i * 16, 16)].astype(jnp.int32)
        w16 = wkb[pl.ds(i * 16, 16)]
        tid16 = tidb[pl.ds(i * 16, 16)].astype(jnp.int32)
        pltpu.async_copy(v_hbm.at[idx16], stage, sem).wait()
        for j in range(16):
            wj = jnp.sum(jnp.where(lane == j, w16, 0.0))
            for k in range(8):
                stage[j, pl.ds(k * 16, 16)] = stage[j, pl.ds(k * 16, 16)] * wj
        pltpu.sync_copy(stage, shared.at[tid16], add=True)
        return 0
    lax.fori_loop(0, nchunk, _chunk, 0)

    plsc.subcore_barrier()
    pltpu.sync_copy(shared.at[pl.ds(lbase, _TPW)],
                    out_hbm.at[pl.ds(gbase, _TPW)])


def _stage2(nfs2, vrows):
    mesh = plsc.VectorSubcoreMesh(core_axis_name="c", subcore_axis_name="s")
    f = functools.partial(
        pl.kernel,
        mesh=mesh,
        compiler_params=pltpu.CompilerParams(needs_layout_passes=False),
        out_type=jax.ShapeDtypeStruct((_N, _D), jnp.float32),
        scratch_types=[
            pltpu.VMEM((_TPW * _R,), jnp.float32),
            pltpu.VMEM((_CAP,), jnp.float32),
            pltpu.VMEM((_CAP,), jnp.float32),
            pltpu.VMEM((_CAP,), jnp.float32),
            pltpu.VMEM((16, _D), jnp.float32),
            pltpu.VMEM((16, _D), jnp.float32),
            pltpu.VMEM_SHARED((_N // 2, _D), jnp.float32),
            pltpu.SemaphoreType.DMA,
        ],
    )
    return f(_sc_body)(nfs2.reshape(_N * _R), vrows)


def _transpose_body(p_ref, out_ref):
    out_ref[0] = p_ref[0].T


def _stage3(pred_rows, B, T):
    p3 = pred_rows.reshape(B, T, _D)
    return pl.pallas_call(
        _transpose_body,
        grid=(B, T // 512),
        in_specs=[pl.BlockSpec((1, 512, _D), lambda b, t: (b, t, 0))],
        out_specs=pl.BlockSpec((1, _D, 512), lambda b, t: (b, 0, t)),
        out_shape=jax.ShapeDtypeStruct((B, _D, T), jnp.float32),
    )(p3)


def kernel(queries, keys, values, sigma):
    B, T = queries.shape[0], queries.shape[1]
    qflat = queries.reshape(B * T, 1, _D)
    kw = keys.reshape(1, _RD)
    iw = (1.0 / (sigma + _EPS)).reshape(1, _RD)
    vrows = values.reshape(B * T * _R, _D)

    nfs2, loss = _stage1(qflat, kw, iw)
    pred_rows = _stage2(nfs2, vrows)
    pred = _stage3(pred_rows, B, T)
    return pred, loss.reshape(())


# speedup vs baseline: 3.0312x; 1.0066x over previous
import functools

import jax
import jax.numpy as jnp
import numpy as np
from jax import lax
from jax.experimental import pallas as pl
from jax.experimental.pallas import tpu as pltpu
from jax.experimental.pallas import tpu_sc as plsc

_D = 128
_R = 64
_TAU_0 = 0.75
_EPS = float(np.finfo(np.float32).tiny)
_TAU = _TAU_0 * (1.0 / (_R + _EPS)) ** (1.0 / _D)

_TB = 512
_RD = _R * _D
_N = 4 * 2048
_NW = 32
_TPW = _N // _NW
_CAP = _TPW * _R + 32


def _weights_body(q_ref, kw_ref, iw_ref, out_ref, loss_ref, imp_ref):
    step = pl.program_id(0)
    nsteps = pl.num_programs(0)

    q = q_ref[:, 0, :]
    qw = jnp.concatenate([q] * _R, axis=1)
    dist = (qw - kw_ref[...]) * iw_ref[...]
    dsq = dist * dist
    cols = [jnp.sum(dsq[:, r * _D:(r + 1) * _D], axis=1, keepdims=True)
            for r in range(_R)]
    d2 = jnp.concatenate(cols, axis=1)
    fire = jnp.exp(d2 * (-0.5 / _D)) + _EPS
    nfs = fire / (jnp.sum(fire, axis=1, keepdims=True) + _EPS)

    excess = jnp.zeros_like(nfs)
    for j in range(_R):
        cj = nfs[:, j:j + 1]
        excess = excess + jnp.where(cj > nfs, cj, 0.0)
    keep = (excess < _TAU).astype(jnp.float32)
    kept = nfs * keep
    nfs2 = kept / (jnp.sum(kept, axis=1, keepdims=True) + _EPS)
    out_ref[...] = nfs2

    blk_imp = jnp.sum(nfs2, axis=0, keepdims=True)

    @pl.when(step == 0)
    def _():
        imp_ref[...] = blk_imp

    @pl.when(step != 0)
    def _():
        imp_ref[...] = imp_ref[...] + blk_imp

    @pl.when(step == nsteps - 1)
    def _():
        imp = imp_ref[...]
        mean = jnp.sum(imp, axis=1, keepdims=True) / _R
        var = jnp.sum((imp - mean) ** 2, axis=1, keepdims=True) / (_R - 1)
        loss_ref[...] = var / (mean * mean + 1e-10)


def _stage1(qflat, kw, iw):
    const = lambda t: (0, 0)
    return pl.pallas_call(
        _weights_body,
        grid=(_N // _TB,),
        in_specs=[
            pl.BlockSpec((_TB, 1, _D), lambda t: (t, 0, 0)),
            pl.BlockSpec((1, _RD), const),
            pl.BlockSpec((1, _RD), const),
        ],
        out_specs=[
            pl.BlockSpec((_TB, _R), lambda t: (t, 0)),
            pl.BlockSpec((1, 1), const),
        ],
        out_shape=[
            jax.ShapeDtypeStruct((_N, _R), jnp.float32),
            jax.ShapeDtypeStruct((1, 1), jnp.float32),
        ],
        scratch_shapes=[pltpu.VMEM((1, _R), jnp.float32)],
    )(qflat, kw, iw)


def _sc_body(w_hbm, v_hbm, out_hbm,
             wv, idxb, wkb, tidb, stage, stage2, zbuf, shared, sem, sem2):
    c = lax.axis_index("c")
    s = lax.axis_index("s")
    wid = c * 16 + s
    gbase = wid * _TPW
    lbase = s * _TPW

    pltpu.sync_copy(w_hbm.at[pl.ds(gbase * _R, _TPW * _R)], wv)

    for j in range(16):
        for k in range(8):
            zbuf[j, pl.ds(k * 16, 16)] = jnp.zeros((16,), jnp.float32)
    for blk in range(16):
        pltpu.sync_copy(zbuf, shared.at[pl.ds(lbase + blk * 16, 16)])

    lane = lax.iota(jnp.int32, 16)

    def _build(t, cnt):
        rowbase = (gbase + t) * _R
        for j in range(4):
            w = wv[pl.ds(t * _R + j * 16, 16)]
            m = w > 0.0
            idxv = rowbase + j * 16 + lane
            tidv = jnp.full((16,), lbase, jnp.int32) + t
            key = jnp.where(m, lane, lane + 16)
            idxb[pl.ds(cnt, 16)] = plsc.sort_key_val(key, idxv.astype(jnp.float32))[1]
            wkb[pl.ds(cnt, 16)] = plsc.sort_key_val(key, w)[1]
            tidb[pl.ds(cnt, 16)] = plsc.sort_key_val(key, tidv.astype(jnp.float32))[1]
            cnt = cnt + jnp.sum(m.astype(jnp.int32))
        return cnt
    cnt = lax.fori_loop(0, _TPW, _build, jnp.int32(0))

    for p in range(2):
        idxb[pl.ds(cnt + p * 16, 16)] = jnp.zeros((16,), jnp.float32)
        wkb[pl.ds(cnt + p * 16, 16)] = jnp.zeros((16,), jnp.float32)
        tidb[pl.ds(cnt + p * 16, 16)] = (
            jnp.full((16,), lbase, jnp.int32).astype(jnp.float32))

    npair = (cnt + 31) // 32

    def _pair(p, _):
        i0 = p * 2
        idx0 = idxb[pl.ds(i0 * 16, 16)].astype(jnp.int32)
        idx1 = idxb[pl.ds(i0 * 16 + 16, 16)].astype(jnp.int32)
        cA = pltpu.async_copy(v_hbm.at[idx0], stage, sem)
        cB = pltpu.async_copy(v_hbm.at[idx1], stage2, sem2)
        for half, (st, cp) in enumerate(((stage, cA), (stage2, cB))):
            w16 = wkb[pl.ds(i0 * 16 + half * 16, 16)]
            tid16 = tidb[pl.ds(i0 * 16 + half * 16, 16)].astype(jnp.int32)
            cp.wait()
            for j in range(16):
                wj = jnp.sum(jnp.where(lane == j, w16, 0.0))
                for k in range(8):
                    st[j, pl.ds(k * 16, 16)] = st[j, pl.ds(k * 16, 16)] * wj
            pltpu.sync_copy(st, shared.at[tid16], add=True)
        return 0
    lax.fori_loop(0, npair, _pair, 0)

    plsc.subcore_barrier()
    pltpu.sync_copy(shared.at[pl.ds(lbase, _TPW)],
                    out_hbm.at[pl.ds(gbase, _TPW)])


def _stage2(nfs2, vrows):
    mesh = plsc.VectorSubcoreMesh(core_axis_name="c", subcore_axis_name="s")
    f = functools.partial(
        pl.kernel,
        mesh=mesh,
        compiler_params=pltpu.CompilerParams(needs_layout_passes=False),
        out_type=jax.ShapeDtypeStruct((_N, _D), jnp.float32),
        scratch_types=[
            pltpu.VMEM((_TPW * _R,), jnp.float32),
            pltpu.VMEM((_CAP,), jnp.float32),
            pltpu.VMEM((_CAP,), jnp.float32),
            pltpu.VMEM((_CAP,), jnp.float32),
            pltpu.VMEM((16, _D), jnp.float32),
            pltpu.VMEM((16, _D), jnp.float32),
            pltpu.VMEM((16, _D), jnp.float32),
            pltpu.VMEM_SHARED((_N // 2, _D), jnp.float32),
            pltpu.SemaphoreType.DMA,
            pltpu.SemaphoreType.DMA,
        ],
    )
    return f(_sc_body)(nfs2.reshape(_N * _R), vrows)


def _transpose_body(p_ref, out_ref):
    out_ref[0] = p_ref[0].T


def _stage3(pred_rows, B, T):
    p3 = pred_rows.reshape(B, T, _D)
    return pl.pallas_call(
        _transpose_body,
        grid=(B, T // 512),
        in_specs=[pl.BlockSpec((1, 512, _D), lambda b, t: (b, t, 0))],
        out_specs=pl.BlockSpec((1, _D, 512), lambda b, t: (b, 0, t)),
        out_shape=jax.ShapeDtypeStruct((B, _D, T), jnp.float32),
    )(p3)


def kernel(queries, keys, values, sigma):
    B, T = queries.shape[0], queries.shape[1]
    qflat = queries.reshape(B * T, 1, _D)
    kw = keys.reshape(1, _RD)
    iw = (1.0 / (sigma + _EPS)).reshape(1, _RD)
    vrows = values.reshape(B * T * _R, _D)

    nfs2, loss = _stage1(qflat, kw, iw)
    pred_rows = _stage2(nfs2, vrows)
    pred = _stage3(pred_rows, B, T)
    return pred, loss.reshape(())


# speedup vs baseline: 3.0392x; 1.0027x over previous
import functools

import jax
import jax.numpy as jnp
import numpy as np
from jax import lax
from jax.experimental import pallas as pl
from jax.experimental.pallas import tpu as pltpu
from jax.experimental.pallas import tpu_sc as plsc

_D = 128
_R = 64
_TAU_0 = 0.75
_EPS = float(np.finfo(np.float32).tiny)
_TAU = _TAU_0 * (1.0 / (_R + _EPS)) ** (1.0 / _D)

_TB = 1024
_RD = _R * _D
_N = 4 * 2048
_NW = 32
_TPW = _N // _NW
_CAP = _TPW * _R + 32


def _weights_body(q_ref, kw_ref, iw_ref, out_ref, loss_ref, imp_ref):
    step = pl.program_id(0)
    nsteps = pl.num_programs(0)

    q = q_ref[:, 0, :]
    qw = jnp.concatenate([q] * _R, axis=1)
    dist = (qw - kw_ref[...]) * iw_ref[...]
    dsq = dist * dist
    cols = [jnp.sum(dsq[:, r * _D:(r + 1) * _D], axis=1, keepdims=True)
            for r in range(_R)]
    d2 = jnp.concatenate(cols, axis=1)
    fire = jnp.exp(d2 * (-0.5 / _D)) + _EPS
    nfs = fire / (jnp.sum(fire, axis=1, keepdims=True) + _EPS)

    excess = jnp.zeros_like(nfs)
    for j in range(_R):
        cj = nfs[:, j:j + 1]
        excess = excess + jnp.where(cj > nfs, cj, 0.0)
    keep = (excess < _TAU).astype(jnp.float32)
    kept = nfs * keep
    nfs2 = kept / (jnp.sum(kept, axis=1, keepdims=True) + _EPS)
    out_ref[...] = nfs2

    blk_imp = jnp.sum(nfs2, axis=0, keepdims=True)

    @pl.when(step == 0)
    def _():
        imp_ref[...] = blk_imp

    @pl.when(step != 0)
    def _():
        imp_ref[...] = imp_ref[...] + blk_imp

    @pl.when(step == nsteps - 1)
    def _():
        imp = imp_ref[...]
        mean = jnp.sum(imp, axis=1, keepdims=True) / _R
        var = jnp.sum((imp - mean) ** 2, axis=1, keepdims=True) / (_R - 1)
        loss_ref[...] = var / (mean * mean + 1e-10)


def _stage1(qflat, kw, iw):
    const = lambda t: (0, 0)
    return pl.pallas_call(
        _weights_body,
        grid=(_N // _TB,),
        in_specs=[
            pl.BlockSpec((_TB, 1, _D), lambda t: (t, 0, 0)),
            pl.BlockSpec((1, _RD), const),
            pl.BlockSpec((1, _RD), const),
        ],
        out_specs=[
            pl.BlockSpec((_TB, _R), lambda t: (t, 0)),
            pl.BlockSpec((1, 1), const),
        ],
        out_shape=[
            jax.ShapeDtypeStruct((_N, _R), jnp.float32),
            jax.ShapeDtypeStruct((1, 1), jnp.float32),
        ],
        scratch_shapes=[pltpu.VMEM((1, _R), jnp.float32)],
    )(qflat, kw, iw)


def _sc_body(w_hbm, v_hbm, out_hbm,
             wv, idxb, wkb, tidb, stage, stage2, zbuf, shared, sem, sem2):
    c = lax.axis_index("c")
    s = lax.axis_index("s")
    wid = c * 16 + s
    gbase = wid * _TPW
    lbase = s * _TPW

    pltpu.sync_copy(w_hbm.at[pl.ds(gbase * _R, _TPW * _R)], wv)

    for j in range(16):
        for k in range(8):
            zbuf[j, pl.ds(k * 16, 16)] = jnp.zeros((16,), jnp.float32)
    for blk in range(16):
        pltpu.sync_copy(zbuf, shared.at[pl.ds(lbase + blk * 16, 16)])

    lane = lax.iota(jnp.int32, 16)

    def _build(t, cnt):
        rowbase = (gbase + t) * _R
        for j in range(4):
            w = wv[pl.ds(t * _R + j * 16, 16)]
            m = w > 0.0
            idxv = rowbase + j * 16 + lane
            tidv = jnp.full((16,), lbase, jnp.int32) + t
            key = jnp.where(m, lane, lane + 16)
            idxb[pl.ds(cnt, 16)] = plsc.sort_key_val(key, idxv.astype(jnp.float32))[1]
            wkb[pl.ds(cnt, 16)] = plsc.sort_key_val(key, w)[1]
            tidb[pl.ds(cnt, 16)] = plsc.sort_key_val(key, tidv.astype(jnp.float32))[1]
            cnt = cnt + jnp.sum(m.astype(jnp.int32))
        return cnt
    cnt = lax.fori_loop(0, _TPW, _build, jnp.int32(0))

    for p in range(2):
        idxb[pl.ds(cnt + p * 16, 16)] = jnp.zeros((16,), jnp.float32)
        wkb[pl.ds(cnt + p * 16, 16)] = jnp.zeros((16,), jnp.float32)
        tidb[pl.ds(cnt + p * 16, 16)] = (
            jnp.full((16,), lbase, jnp.int32).astype(jnp.float32))

    npair = (cnt + 31) // 32

    def _pair(p, _):
        i0 = p * 2
        idx0 = idxb[pl.ds(i0 * 16, 16)].astype(jnp.int32)
        idx1 = idxb[pl.ds(i0 * 16 + 16, 16)].astype(jnp.int32)
        cA = pltpu.async_copy(v_hbm.at[idx0], stage, sem)
        cB = pltpu.async_copy(v_hbm.at[idx1], stage2, sem2)
        for half, (st, cp) in enumerate(((stage, cA), (stage2, cB))):
            w16 = wkb[pl.ds(i0 * 16 + half * 16, 16)]
            tid16 = tidb[pl.ds(i0 * 16 + half * 16, 16)].astype(jnp.int32)
            cp.wait()
            for j in range(16):
                wj = jnp.sum(jnp.where(lane == j, w16, 0.0))
                for k in range(8):
                    st[j, pl.ds(k * 16, 16)] = st[j, pl.ds(k * 16, 16)] * wj
            pltpu.sync_copy(st, shared.at[tid16], add=True)
        return 0
    lax.fori_loop(0, npair, _pair, 0)

    plsc.subcore_barrier()
    pltpu.sync_copy(shared.at[pl.ds(lbase, _TPW)],
                    out_hbm.at[pl.ds(gbase, _TPW)])


def _stage2(nfs2, vrows):
    mesh = plsc.VectorSubcoreMesh(core_axis_name="c", subcore_axis_name="s")
    f = functools.partial(
        pl.kernel,
        mesh=mesh,
        compiler_params=pltpu.CompilerParams(needs_layout_passes=False),
        out_type=jax.ShapeDtypeStruct((_N, _D), jnp.float32),
        scratch_types=[
            pltpu.VMEM((_TPW * _R,), jnp.float32),
            pltpu.VMEM((_CAP,), jnp.float32),
            pltpu.VMEM((_CAP,), jnp.float32),
            pltpu.VMEM((_CAP,), jnp.float32),
            pltpu.VMEM((16, _D), jnp.float32),
            pltpu.VMEM((16, _D), jnp.float32),
            pltpu.VMEM((16, _D), jnp.float32),
            pltpu.VMEM_SHARED((_N // 2, _D), jnp.float32),
            pltpu.SemaphoreType.DMA,
            pltpu.SemaphoreType.DMA,
        ],
    )
    return f(_sc_body)(nfs2.reshape(_N * _R), vrows)


def _transpose_body(p_ref, out_ref):
    out_ref[0] = p_ref[0].T


def _stage3(pred_rows, B, T):
    p3 = pred_rows.reshape(B, T, _D)
    return pl.pallas_call(
        _transpose_body,
        grid=(B, T // 512),
        in_specs=[pl.BlockSpec((1, 512, _D), lambda b, t: (b, t, 0))],
        out_specs=pl.BlockSpec((1, _D, 512), lambda b, t: (b, 0, t)),
        out_shape=jax.ShapeDtypeStruct((B, _D, T), jnp.float32),
    )(p3)


def kernel(queries, keys, values, sigma):
    B, T = queries.shape[0], queries.shape[1]
    qflat = queries.reshape(B * T, 1, _D)
    kw = keys.reshape(1, _RD)
    iw = (1.0 / (sigma + _EPS)).reshape(1, _RD)
    vrows = values.reshape(B * T * _R, _D)

    nfs2, loss = _stage1(qflat, kw, iw)
    pred_rows = _stage2(nfs2, vrows)
    pred = _stage3(pred_rows, B, T)
    return pred, loss.reshape(())
